# Initial kernel scaffold; baseline (speedup 1.0000x reference)
import jax
import jax.numpy as jnp
from jax.experimental import pallas as pl


def kernel(x, seg, tok_table, pos_table, seg_table, gamma, beta):
    raise NotImplementedError("write your pallas kernel here")


# speedup vs baseline: 1.6256x; 1.6256x over previous
import jax
import jax.numpy as jnp
from jax import lax
from jax.experimental import pallas as pl
from jax.experimental.pallas import tpu as pltpu
from jax.experimental.pallas import tpu_sc as plsc

NC = 2
NS = 16
NW = NC * NS
L = 16

SJ = 2
SC_ = 100
S = SJ * SC_
D = 64
KD = D // L

_EPS = 1e-5
_MAGIC = 0x5F3759DF

_DNUMS = lax.GatherDimensionNumbers(
    offset_dims=(), collapsed_slice_dims=(0,), start_index_map=(0,))


def _bcast_lane(vec, lane):
  idx = jnp.full((L, 1), lane, jnp.int32)
  return lax.gather(vec, idx, dimension_numbers=_DNUMS, slice_sizes=(1,),
                    mode=lax.GatherScatterMode.PROMISE_IN_BOUNDS)


def _body(x_hbm, segf_hbm, tok_hbm, pos_hbm, segtab_hbm, g_hbm, b_hbm,
          out_hbm, xidx_v, segv, posv, segtab_v, gv, bv, rows, sem):
  nb = x_hbm.shape[0] // NW
  wid = lax.axis_index("s") * NC + lax.axis_index("c")

  pltpu.sync_copy(pos_hbm, posv)
  pltpu.sync_copy(segtab_hbm, segtab_v)
  pltpu.sync_copy(g_hbm, gv)
  pltpu.sync_copy(b_hbm, bv)

  g = [gv[k] for k in range(KD)]
  bt = [bv[k] for k in range(KD)]
  s0 = [segtab_v[0, pl.ds(k * L, L)] for k in range(KD)]
  sd = [segtab_v[1, pl.ds(k * L, L)] - s0[k] for k in range(KD)]

  def chunk(i, carry):
    b = wid * nb + i
    pltpu.sync_copy(x_hbm.at[b], xidx_v)
    pltpu.sync_copy(segf_hbm.at[b], segv)
    cps = [pltpu.async_copy(tok_hbm.at[xidx_v.at[j]],
                            rows.at[pl.ds(j * SC_, SC_)], sem)
           for j in range(SJ)]
    for cp in cps:
      cp.wait()

    def rowbody(r, c2):
      goff = (r // L) * L
      sgroup = segv[pl.ds(goff, L)]
      m = _bcast_lane(sgroup, r - goff)
      vs = []
      for k in range(KD):
        v = rows[r, pl.ds(k * L, L)] + posv[r, pl.ds(k * L, L)]
        v = v + (m * sd[k] + s0[k])
        vs.append(v)
      t = (vs[0] + vs[1]) + (vs[2] + vs[3])
      u = ((vs[0] * vs[0] + vs[1] * vs[1])
           + (vs[2] * vs[2] + vs[3] * vs[3]))
      mean = jnp.full((L,), jnp.sum(t)) * (1.0 / D)
      ex2 = jnp.full((L,), jnp.sum(u)) * (1.0 / D)
      var = ex2 - mean * mean + _EPS
      iv = plsc.bitcast(var, jnp.int32)
      iv = jnp.full((L,), _MAGIC, jnp.int32) - lax.shift_right_logical(iv, 1)
      y = plsc.bitcast(iv, jnp.float32)
      for _ in range(3):
        y = y * (1.5 - 0.5 * var * y * y)
      for k in range(KD):
        o = (vs[k] - mean) * (y * g[k]) + bt[k]
        rows[r, pl.ds(k * L, L)] = o
      return c2

    lax.fori_loop(0, S, rowbody, 0)
    pltpu.sync_copy(rows, out_hbm.at[b])
    return carry

  lax.fori_loop(0, nb, chunk, 0)


def kernel(x, seg, tok_table, pos_table, seg_table, gamma, beta):
  B, seq = x.shape
  d = tok_table.shape[1]
  x3 = x.reshape(B, SJ, SC_)
  segf = jnp.pad(seg.astype(jnp.float32), ((0, 0), (0, 256 - seq)))
  g2 = gamma.reshape(KD, L)
  b2 = beta.reshape(KD, L)

  mesh = plsc.VectorSubcoreMesh(core_axis_name="c", subcore_axis_name="s")
  fn = pl.kernel(
      _body,
      out_type=jax.ShapeDtypeStruct((B, seq, d), jnp.float32),
      mesh=mesh,
      compiler_params=pltpu.CompilerParams(
          needs_layout_passes=False, use_tc_tiling_on_sc=False),
      scratch_types=[
          pltpu.VMEM((SJ, SC_), jnp.int32),
          pltpu.VMEM((256,), jnp.float32),
          pltpu.VMEM((S, d), jnp.float32),
          pltpu.VMEM((2, d), jnp.float32),
          pltpu.VMEM((KD, L), jnp.float32),
          pltpu.VMEM((KD, L), jnp.float32),
          pltpu.VMEM((S, d), jnp.float32),
          pltpu.SemaphoreType.DMA,
      ],
  )
  return fn(x3, segf, tok_table, pos_table, seg_table, g2, b2)


# speedup vs baseline: 1.6260x; 1.0002x over previous
import jax
import jax.numpy as jnp
from jax import lax
from jax.experimental import pallas as pl
from jax.experimental.pallas import tpu as pltpu
from jax.experimental.pallas import tpu_sc as plsc

NC = 2
NS = 16
NW = NC * NS
L = 16

SJ = 2
SC_ = 100
S = SJ * SC_
D = 64
KD = D // L

_EPS = 1e-5
_MAGIC = 0x5F3759DF

_DNUMS = lax.GatherDimensionNumbers(
    offset_dims=(), collapsed_slice_dims=(0,), start_index_map=(0,))


def _bcast_lane(vec, lane):
  idx = jnp.full((L, 1), lane, jnp.int32)
  return lax.gather(vec, idx, dimension_numbers=_DNUMS, slice_sizes=(1,),
                    mode=lax.GatherScatterMode.PROMISE_IN_BOUNDS)


def _body(x_hbm, ctl_hbm, tok_hbm, pos_hbm, segtab_hbm, g_hbm, b_hbm,
          out_hbm, xidx_v, ctlv, posv, segtab_v, gv, bv, rows, ostg, sem):
  nb = x_hbm.shape[0] // NW
  wid = lax.axis_index("s") * NC + lax.axis_index("c")

  pltpu.sync_copy(pos_hbm, posv)
  pltpu.sync_copy(segtab_hbm, segtab_v)
  pltpu.sync_copy(g_hbm, gv)
  pltpu.sync_copy(b_hbm, bv)

  g = [gv[k] for k in range(KD)]
  bt = [bv[k] for k in range(KD)]
  s0 = [segtab_v[0, pl.ds(k * L, L)] for k in range(KD)]
  sd = [segtab_v[1, pl.ds(k * L, L)] - s0[k] for k in range(KD)]

  def chunk(i, carry):
    b = wid * nb + i
    pltpu.sync_copy(x_hbm.at[b], xidx_v)
    pltpu.sync_copy(ctl_hbm.at[b], ctlv)
    cps = [pltpu.async_copy(tok_hbm.at[xidx_v.at[j]],
                            rows.at[pl.ds(j * SC_, SC_)], sem)
           for j in range(SJ)]
    for cp in cps:
      cp.wait()

    def rowbody(r, c2):
      goff = (r // L) * L
      cgroup = ctlv[pl.ds(goff, L)]
      m = _bcast_lane(cgroup, r - goff)
      par = m > 1.5
      ms = jnp.where(par, m - 2.0, m)
      vs = []
      for k in range(KD):
        lo = rows[r, pl.ds(k * L, L)]
        hi = rows[r, pl.ds(D + k * L, L)]
        v = jnp.where(par, hi, lo) + posv[r, pl.ds(k * L, L)]
        v = v + (ms * sd[k] + s0[k])
        vs.append(v)
      t = (vs[0] + vs[1]) + (vs[2] + vs[3])
      u = ((vs[0] * vs[0] + vs[1] * vs[1])
           + (vs[2] * vs[2] + vs[3] * vs[3]))
      mean = jnp.full((L,), jnp.sum(t)) * (1.0 / D)
      ex2 = jnp.full((L,), jnp.sum(u)) * (1.0 / D)
      var = ex2 - mean * mean + _EPS
      iv = plsc.bitcast(var, jnp.int32)
      iv = jnp.full((L,), _MAGIC, jnp.int32) - lax.shift_right_logical(iv, 1)
      y = plsc.bitcast(iv, jnp.float32)
      for _ in range(3):
        y = y * (1.5 - 0.5 * var * y * y)
      r2 = r // 2
      half = (r - r2 * 2) * D
      for k in range(KD):
        o = (vs[k] - mean) * (y * g[k]) + bt[k]
        ostg[r2, pl.ds(half + k * L, L)] = o
      return c2

    lax.fori_loop(0, S, rowbody, 0)
    pltpu.sync_copy(ostg, out_hbm.at[b])
    return carry

  lax.fori_loop(0, nb, chunk, 0)


def kernel(x, seg, tok_table, pos_table, seg_table, gamma, beta):
  B, seq = x.shape
  d = tok_table.shape[1]
  v2 = tok_table.shape[0] // 2
  tok2 = tok_table.reshape(v2, 2 * d)
  xp = (x >> 1).reshape(B, SJ, SC_)
  ctl = jnp.pad(((x & 1) * 2 + seg).astype(jnp.float32),
                ((0, 0), (0, 256 - seq)))
  g2 = gamma.reshape(KD, L)
  b2 = beta.reshape(KD, L)

  mesh = plsc.VectorSubcoreMesh(core_axis_name="c", subcore_axis_name="s")
  fn = pl.kernel(
      _body,
      out_type=jax.ShapeDtypeStruct((B, seq // 2, 2 * d), jnp.float32),
      mesh=mesh,
      compiler_params=pltpu.CompilerParams(
          needs_layout_passes=False, use_tc_tiling_on_sc=False),
      scratch_types=[
          pltpu.VMEM((SJ, SC_), jnp.int32),
          pltpu.VMEM((256,), jnp.float32),
          pltpu.VMEM((S, d), jnp.float32),
          pltpu.VMEM((2, d), jnp.float32),
          pltpu.VMEM((KD, L), jnp.float32),
          pltpu.VMEM((KD, L), jnp.float32),
          pltpu.VMEM((S, 2 * d), jnp.float32),
          pltpu.VMEM((S // 2, 2 * d), jnp.float32),
          pltpu.SemaphoreType.DMA,
      ],
  )
  out = fn(xp, ctl, tok2, pos_table, seg_table, g2, b2)
  return out.reshape(B, seq, d)


# speedup vs baseline: 1.7389x; 1.0694x over previous
import jax
import jax.numpy as jnp
from jax import lax
from jax.experimental import pallas as pl
from jax.experimental.pallas import tpu as pltpu
from jax.experimental.pallas import tpu_sc as plsc

NC = 2
NS = 16
NW = NC * NS
L = 16

SJ = 2
SC_ = 100
S = SJ * SC_
D = 64
KD = D // L

_EPS = 1e-5
_MAGIC = 0x5F3759DF

_DNUMS = lax.GatherDimensionNumbers(
    offset_dims=(), collapsed_slice_dims=(0,), start_index_map=(0,))


def _bcast_lane(vec, lane):
  idx = jnp.full((L, 1), lane, jnp.int32)
  return lax.gather(vec, idx, dimension_numbers=_DNUMS, slice_sizes=(1,),
                    mode=lax.GatherScatterMode.PROMISE_IN_BOUNDS)


def _body(x_hbm, seg_hbm, tok_hbm, pos_hbm, segtab_hbm, g_hbm, b_hbm,
          out_hbm, xidx_v, segv, posv, segtab_v, gv, bv,
          rows0, rows1, sem_g0, sem_g1, sem_o0, sem_o1):
  nb = x_hbm.shape[0] // NW
  wid = lax.axis_index("s") * NC + lax.axis_index("c")
  b0 = wid * nb

  pltpu.sync_copy(pos_hbm, posv)
  pltpu.sync_copy(segtab_hbm, segtab_v)
  pltpu.sync_copy(g_hbm, gv)
  pltpu.sync_copy(b_hbm, bv)
  pltpu.sync_copy(x_hbm.at[pl.ds(b0, nb)], xidx_v)
  pltpu.sync_copy(seg_hbm.at[pl.ds(b0, nb)], segv)

  g = [gv[k] for k in range(KD)]
  bt = [bv[k] for k in range(KD)]
  s0 = [segtab_v[0, pl.ds(k * L, L)] for k in range(KD)]
  sd = [segtab_v[1, pl.ds(k * L, L)] - s0[k] for k in range(KD)]

  rows_sl = (rows0, rows1)
  sem_g = (sem_g0, sem_g1)
  sem_o = (sem_o0, sem_o1)

  def start_gather(i, slot):
    for j in range(SJ):
      pltpu.async_copy(tok_hbm.at[xidx_v.at[i, j]],
                       rows_sl[slot].at[pl.ds(j * SC_, SC_)], sem_g[slot])

  def wait_gather(i, slot):
    for j in range(SJ):
      pltpu.make_async_copy(tok_hbm.at[xidx_v.at[i, j]],
                            rows_sl[slot].at[pl.ds(j * SC_, SC_)],
                            sem_g[slot]).wait()

  def start_out(i, slot):
    pltpu.async_copy(rows_sl[slot], out_hbm.at[b0 + i], sem_o[slot])

  def wait_out(i, slot):
    pltpu.make_async_copy(rows_sl[slot], out_hbm.at[b0 + i],
                          sem_o[slot]).wait()

  def compute(i, slot):
    rows = rows_sl[slot]

    def rowbody(r, c2):
      goff = (r // L) * L
      sgroup = segv[i, pl.ds(goff, L)]
      m = _bcast_lane(sgroup, r - goff)
      vs = []
      for k in range(KD):
        v = rows[r, pl.ds(k * L, L)] + posv[r, pl.ds(k * L, L)]
        v = v + (m * sd[k] + s0[k])
        vs.append(v)
      t = (vs[0] + vs[1]) + (vs[2] + vs[3])
      u = ((vs[0] * vs[0] + vs[1] * vs[1])
           + (vs[2] * vs[2] + vs[3] * vs[3]))
      mean = jnp.full((L,), jnp.sum(t)) * (1.0 / D)
      ex2 = jnp.full((L,), jnp.sum(u)) * (1.0 / D)
      var = ex2 - mean * mean + _EPS
      iv = plsc.bitcast(var, jnp.int32)
      iv = jnp.full((L,), _MAGIC, jnp.int32) - lax.shift_right_logical(iv, 1)
      y = plsc.bitcast(iv, jnp.float32)
      for _ in range(3):
        y = y * (1.5 - 0.5 * var * y * y)
      for k in range(KD):
        o = (vs[k] - mean) * (y * g[k]) + bt[k]
        rows[r, pl.ds(k * L, L)] = o
      return c2

    lax.fori_loop(0, S, rowbody, 0)

  start_gather(0, 0)

  def iter2(it2, carry):
    for s2 in range(2):
      i = it2 * 2 + s2

      @pl.when(i + 1 < nb)
      def _():
        @pl.when(i >= 1)
        def _():
          wait_out(i - 1, 1 - s2)
        start_gather(i + 1, 1 - s2)

      wait_gather(i, s2)
      compute(i, s2)
      start_out(i, s2)
    return carry

  lax.fori_loop(0, nb // 2, iter2, 0)
  wait_out(nb - 1, 1)


def kernel(x, seg, tok_table, pos_table, seg_table, gamma, beta):
  B, seq = x.shape
  d = tok_table.shape[1]
  x3 = x.reshape(B, SJ, SC_)
  segf = jnp.pad(seg.astype(jnp.float32), ((0, 0), (0, 256 - seq)))
  g2 = gamma.reshape(KD, L)
  b2 = beta.reshape(KD, L)
  nb = B // NW

  mesh = plsc.VectorSubcoreMesh(core_axis_name="c", subcore_axis_name="s")
  fn = pl.kernel(
      _body,
      out_type=jax.ShapeDtypeStruct((B, seq, d), jnp.float32),
      mesh=mesh,
      compiler_params=pltpu.CompilerParams(
          needs_layout_passes=False, use_tc_tiling_on_sc=False),
      scratch_types=[
          pltpu.VMEM((nb, SJ, SC_), jnp.int32),
          pltpu.VMEM((nb, 256), jnp.float32),
          pltpu.VMEM((S, d), jnp.float32),
          pltpu.VMEM((2, d), jnp.float32),
          pltpu.VMEM((KD, L), jnp.float32),
          pltpu.VMEM((KD, L), jnp.float32),
          pltpu.VMEM((S, d), jnp.float32),
          pltpu.VMEM((S, d), jnp.float32),
          pltpu.SemaphoreType.DMA,
          pltpu.SemaphoreType.DMA,
          pltpu.SemaphoreType.DMA,
          pltpu.SemaphoreType.DMA,
      ],
  )
  return fn(x3, segf, tok_table, pos_table, seg_table, g2, b2)


# speedup vs baseline: 2.1663x; 1.2458x over previous
import jax
import jax.numpy as jnp
from jax import lax
from jax.experimental import pallas as pl
from jax.experimental.pallas import tpu as pltpu
from jax.experimental.pallas import tpu_sc as plsc

NC = 2
NS = 16
NW = NC * NS
L = 16

SJ = 2
SC_ = 100
S = SJ * SC_
D = 64
KD = D // L

_EPS = 1e-5
_MAGIC = 0x5F3759DF

_DNUMS = lax.GatherDimensionNumbers(
    offset_dims=(), collapsed_slice_dims=(0,), start_index_map=(0,))


def _bcast_lane(vec, lane):
  idx = jnp.full((L, 1), lane, jnp.int32)
  return lax.gather(vec, idx, dimension_numbers=_DNUMS, slice_sizes=(1,),
                    mode=lax.GatherScatterMode.PROMISE_IN_BOUNDS)


def _body(x_hbm, seg_hbm, tok_hbm, pos_hbm, segtab_hbm, g_hbm, b_hbm,
          out_hbm, xidx_v, segv, posv, segtab_v, gv, bv,
          rows0, rows1, ostg0, ostg1, sem_g0, sem_g1, sem_o0, sem_o1):
  nb = x_hbm.shape[0] // NW
  wid = lax.axis_index("s") * NC + lax.axis_index("c")
  b0 = wid * nb

  pltpu.sync_copy(pos_hbm, posv)
  pltpu.sync_copy(segtab_hbm, segtab_v)
  pltpu.sync_copy(g_hbm, gv)
  pltpu.sync_copy(b_hbm, bv)
  pltpu.sync_copy(x_hbm.at[pl.ds(b0, nb)], xidx_v)
  pltpu.sync_copy(seg_hbm.at[pl.ds(b0, nb)], segv)

  g = [gv[k] for k in range(KD)]
  bt = [bv[k] for k in range(KD)]
  s0 = [segtab_v[0, pl.ds(k * L, L)] for k in range(KD)]
  sd = [segtab_v[1, pl.ds(k * L, L)] - s0[k] for k in range(KD)]

  rows_sl = (rows0, rows1)
  ostg_sl = (ostg0, ostg1)
  sem_g = (sem_g0, sem_g1)
  sem_o = (sem_o0, sem_o1)

  def start_gather(i, slot):
    for j in range(SJ):
      pltpu.async_copy(tok_hbm.at[xidx_v.at[i, j]],
                       rows_sl[slot].at[pl.ds(j * SC_, SC_)], sem_g[slot])

  def wait_gather(i, slot):
    for j in range(SJ):
      pltpu.make_async_copy(tok_hbm.at[xidx_v.at[i, j]],
                            rows_sl[slot].at[pl.ds(j * SC_, SC_)],
                            sem_g[slot]).wait()

  def start_out(i, slot):
    pltpu.async_copy(ostg_sl[slot], out_hbm.at[b0 + i], sem_o[slot])

  def wait_out(i, slot):
    pltpu.make_async_copy(ostg_sl[slot], out_hbm.at[b0 + i],
                          sem_o[slot]).wait()

  def compute(i, slot):
    rows = rows_sl[slot]
    ostg = ostg_sl[slot]

    @plsc.parallel_loop(0, S, unroll=2)
    def rowbody(r):
      goff = (r // L) * L
      sgroup = segv[i, pl.ds(goff, L)]
      m = _bcast_lane(sgroup, r - goff)
      vs = []
      for k in range(KD):
        v = rows[r, pl.ds(k * L, L)] + posv[r, pl.ds(k * L, L)]
        v = v + (m * sd[k] + s0[k])
        vs.append(v)
      t = (vs[0] + vs[1]) + (vs[2] + vs[3])
      u = ((vs[0] * vs[0] + vs[1] * vs[1])
           + (vs[2] * vs[2] + vs[3] * vs[3]))
      mean = jnp.full((L,), jnp.sum(t)) * (1.0 / D)
      ex2 = jnp.full((L,), jnp.sum(u)) * (1.0 / D)
      var = ex2 - mean * mean + _EPS
      iv = plsc.bitcast(var, jnp.int32)
      iv = jnp.full((L,), _MAGIC, jnp.int32) - lax.shift_right_logical(iv, 1)
      y = plsc.bitcast(iv, jnp.float32)
      for _ in range(3):
        y = y * (1.5 - 0.5 * var * y * y)
      for k in range(KD):
        o = (vs[k] - mean) * (y * g[k]) + bt[k]
        ostg[r, pl.ds(k * L, L)] = o

  start_gather(0, 0)

  def iter2(it2, carry):
    for s2 in range(2):
      i = it2 * 2 + s2

      @pl.when(i + 1 < nb)
      def _():
        start_gather(i + 1, 1 - s2)

      wait_gather(i, s2)

      @pl.when(i >= 2)
      def _():
        wait_out(i - 2, s2)

      compute(i, s2)
      start_out(i, s2)
    return carry

  lax.fori_loop(0, nb // 2, iter2, 0)
  wait_out(nb - 2, 0)
  wait_out(nb - 1, 1)


def kernel(x, seg, tok_table, pos_table, seg_table, gamma, beta):
  B, seq = x.shape
  d = tok_table.shape[1]
  x3 = x.reshape(B, SJ, SC_)
  segf = jnp.pad(seg.astype(jnp.float32), ((0, 0), (0, 256 - seq)))
  g2 = gamma.reshape(KD, L)
  b2 = beta.reshape(KD, L)
  nb = B // NW

  mesh = plsc.VectorSubcoreMesh(core_axis_name="c", subcore_axis_name="s")
  fn = pl.kernel(
      _body,
      out_type=jax.ShapeDtypeStruct((B, seq, d), jnp.float32),
      mesh=mesh,
      compiler_params=pltpu.CompilerParams(
          needs_layout_passes=False, use_tc_tiling_on_sc=False),
      scratch_types=[
          pltpu.VMEM((nb, SJ, SC_), jnp.int32),
          pltpu.VMEM((nb, 256), jnp.float32),
          pltpu.VMEM((S, d), jnp.float32),
          pltpu.VMEM((2, d), jnp.float32),
          pltpu.VMEM((KD, L), jnp.float32),
          pltpu.VMEM((KD, L), jnp.float32),
          pltpu.VMEM((S, d), jnp.float32),
          pltpu.VMEM((S, d), jnp.float32),
          pltpu.VMEM((S, d), jnp.float32),
          pltpu.VMEM((S, d), jnp.float32),
          pltpu.SemaphoreType.DMA,
          pltpu.SemaphoreType.DMA,
          pltpu.SemaphoreType.DMA,
          pltpu.SemaphoreType.DMA,
      ],
  )
  return fn(x3, segf, tok_table, pos_table, seg_table, g2, b2)
